# baseline (speedup 1.0000x reference)
import functools

import jax
import jax.numpy as jnp
from jax.experimental import pallas as pl

DIM = 2048
MOE_INTER = 1408
N_EXPERTS = 16
TOP_K = 2
N_SHARED = 2
N_E_ALL = N_EXPERTS + N_SHARED
INTER_PAD = 1536
NTILE = 512
LANES = 128


def _gate_body(x_ref, gw_ref, w_ref):
    xb = x_ref[...]
    logits = jax.lax.dot_general(
        xb, gw_ref[...], (((1,), (1,)), ((), ())),
        precision=jax.lax.Precision.HIGHEST,
        preferred_element_type=jnp.float32)
    tb = logits.shape[0]
    li = jax.lax.broadcasted_iota(jnp.int32, (tb, LANES), 1)
    valid = li < N_EXPERTS
    logits = jnp.where(valid, logits, -1e30)
    m = jnp.max(logits, axis=1, keepdims=True)
    p = jnp.exp(logits - m)
    p = p / jnp.sum(p, axis=1, keepdims=True)
    w = jnp.zeros((tb, LANES), jnp.float32)
    for i in range(N_EXPERTS):
        pi = p[:, i:i + 1]
        gt = jnp.sum((p > pi).astype(jnp.float32), axis=1, keepdims=True)
        sel = (gt <= 1.0).astype(jnp.float32)
        w = jnp.where(li == i, pi * sel, w)
    w = jnp.where((li >= N_EXPERTS) & (li < N_E_ALL), 1.0, w)
    w_ref[...] = w


def _expert_body(x_ref, wgt_ref, w1_ref, w3_ref, w2_ref, y_ref):
    e = pl.program_id(0)
    n = pl.program_id(1)

    @pl.when((e == 0) & (n == 0))
    def _():
        y_ref[...] = jnp.zeros_like(y_ref)

    xb = x_ref[...]
    t1 = jax.lax.dot_general(
        xb, w1_ref[0], (((1,), (1,)), ((), ())),
        preferred_element_type=jnp.float32)
    t3 = jax.lax.dot_general(
        xb, w3_ref[0], (((1,), (1,)), ((), ())),
        preferred_element_type=jnp.float32)
    h = (t1 / (1.0 + jnp.exp(-t1))) * t3
    hb = h.astype(jnp.bfloat16)
    contrib = jax.lax.dot_general(
        hb, w2_ref[0], (((1,), (1,)), ((), ())),
        preferred_element_type=jnp.float32)
    tb = contrib.shape[0]
    li = jax.lax.broadcasted_iota(jnp.int32, (tb, LANES), 1)
    wsel = jnp.sum(jnp.where(li == e, wgt_ref[...], 0.0), axis=1,
                   keepdims=True)
    y_ref[...] += contrib * wsel


def _run_experts(xb, wgt, w1p, w3p, w2p, tb):
    n_n = INTER_PAD // NTILE
    return pl.pallas_call(
        _expert_body,
        grid=(N_E_ALL, n_n),
        in_specs=[
            pl.BlockSpec((tb, DIM), lambda e, n: (0, 0)),
            pl.BlockSpec((tb, LANES), lambda e, n: (0, 0)),
            pl.BlockSpec((1, NTILE, DIM), lambda e, n: (e, n, 0)),
            pl.BlockSpec((1, NTILE, DIM), lambda e, n: (e, n, 0)),
            pl.BlockSpec((1, DIM, NTILE), lambda e, n: (e, 0, n)),
        ],
        out_specs=pl.BlockSpec((tb, DIM), lambda e, n: (0, 0)),
        out_shape=jax.ShapeDtypeStruct((tb, DIM), jnp.float32),
    )(xb, wgt, w1p, w3p, w2p)


@functools.partial(jax.jit, static_argnames=())
def kernel(x, gate_w, w1, w2, w3, sw1, sw2, sw3):
    shape = x.shape
    xf = x.reshape(-1, DIM)
    n_tok = xf.shape[0]

    gwp = jnp.zeros((LANES, DIM), jnp.float32).at[:N_EXPERTS].set(gate_w)
    wgt = pl.pallas_call(
        _gate_body,
        grid=(4,),
        in_specs=[
            pl.BlockSpec((n_tok // 4, DIM), lambda t: (t, 0)),
            pl.BlockSpec((LANES, DIM), lambda t: (0, 0)),
        ],
        out_specs=pl.BlockSpec((n_tok // 4, LANES), lambda t: (t, 0)),
        out_shape=jax.ShapeDtypeStruct((n_tok, LANES), jnp.float32),
    )(xf, gwp)

    sw1e = sw1.reshape(N_SHARED, MOE_INTER, DIM)
    sw3e = sw3.reshape(N_SHARED, MOE_INTER, DIM)
    sw2e = sw2.reshape(DIM, N_SHARED, MOE_INTER).transpose(1, 0, 2)
    pad_n = INTER_PAD - MOE_INTER
    w1p = jnp.pad(jnp.concatenate([w1, sw1e], 0),
                  ((0, 0), (0, pad_n), (0, 0))).astype(jnp.bfloat16)
    w3p = jnp.pad(jnp.concatenate([w3, sw3e], 0),
                  ((0, 0), (0, pad_n), (0, 0))).astype(jnp.bfloat16)
    w2p = jnp.pad(jnp.concatenate([w2, sw2e], 0),
                  ((0, 0), (0, 0), (0, pad_n))).astype(jnp.bfloat16)

    xb = xf.astype(jnp.bfloat16)
    half = n_tok // 2
    y0 = _run_experts(xb[:half], wgt[:half], w1p, w3p, w2p, half)
    y1 = _run_experts(xb[half:], wgt[half:], w1p, w3p, w2p, half)
    return jnp.concatenate([y0, y1], axis=0).reshape(shape)


# speedup vs baseline: 2.1023x; 2.1023x over previous
import functools

import jax
import jax.numpy as jnp
from jax import lax
from jax.experimental import pallas as pl
from jax.experimental.pallas import tpu as pltpu
from jax.experimental.pallas import tpu_sc as plsc

DIM = 2048
INTER = 1408
NE = 16
NTOK = 4096
NA = 2 * NTOK
BT = 128
NB = NA // BT + NE
P = NB * BT
LANES = 128
GT = 1024
NW = 32
CH = 32


def _gate_body(x_ref, gw_ref, wt_ref, ints_ref, sizes_ref, carry_ref):
    t = pl.program_id(0)

    @pl.when(t == 0)
    def _():
        carry_ref[...] = jnp.zeros_like(carry_ref)

    xb = x_ref[...]
    logits = lax.dot_general(
        xb, gw_ref[...], (((1,), (1,)), ((), ())),
        preferred_element_type=jnp.float32)
    li = lax.broadcasted_iota(jnp.int32, (GT, LANES), 1)
    valid = li < NE
    logits = jnp.where(valid, logits, -1e30)
    m = jnp.max(logits, axis=1, keepdims=True)
    p = jnp.exp(logits - m)
    p = p / jnp.sum(p, axis=1, keepdims=True)
    v1 = jnp.max(p, axis=1, keepdims=True)
    e0 = jnp.min(jnp.where(p == v1, li, LANES), axis=1, keepdims=True)
    p2 = jnp.where(valid & (li != e0), p, -1.0)
    v2 = jnp.max(p2, axis=1, keepdims=True)
    e1 = jnp.min(jnp.where(p2 == v2, li, LANES), axis=1, keepdims=True)
    wt_ref[...] = jnp.where(li == 0, v1, jnp.where(li == 1, v2, 0.0))
    oh0 = (li == e0).astype(jnp.float32)
    oh1 = (li == e1).astype(jnp.float32)
    inc = oh0 + oh1
    ii = lax.broadcasted_iota(jnp.int32, (GT, GT), 0)
    jj = lax.broadcasted_iota(jnp.int32, (GT, GT), 1)
    ltri = (ii > jj).astype(jnp.float32)
    prefix = lax.dot_general(
        ltri, inc, (((1,), (0,)), ((), ())),
        preferred_element_type=jnp.float32)
    prefix = prefix + carry_ref[0:1, :]
    r0 = jnp.sum(jnp.where(li == e0, prefix, 0.0), axis=1, keepdims=True)
    r1 = jnp.sum(jnp.where(li == e1, prefix, 0.0), axis=1, keepdims=True)
    carry_ref[0:1, :] = carry_ref[0:1, :] + jnp.sum(inc, axis=0, keepdims=True)
    ints_ref[...] = jnp.where(
        li == 0, e0,
        jnp.where(li == 1, e1,
                  jnp.where(li == 2, r0.astype(jnp.int32),
                            jnp.where(li == 3, r1.astype(jnp.int32), 0))))

    @pl.when(t == NTOK // GT - 1)
    def _():
        sizes_ref[...] = carry_ref[...]


def _run_gate(xf, gwp):
    return pl.pallas_call(
        _gate_body,
        grid=(NTOK // GT,),
        in_specs=[
            pl.BlockSpec((GT, DIM), lambda t: (t, 0)),
            pl.BlockSpec((LANES, DIM), lambda t: (0, 0)),
        ],
        out_specs=[
            pl.BlockSpec((GT, LANES), lambda t: (t, 0)),
            pl.BlockSpec((GT, LANES), lambda t: (t, 0)),
            pl.BlockSpec((8, LANES), lambda t: (0, 0)),
        ],
        out_shape=[
            jax.ShapeDtypeStruct((NTOK, LANES), jnp.float32),
            jax.ShapeDtypeStruct((NTOK, LANES), jnp.int32),
            jax.ShapeDtypeStruct((8, LANES), jnp.float32),
        ],
        scratch_shapes=[pltpu.VMEM((8, LANES), jnp.float32)],
    )(xf, gwp)


def _dispatch_sc(tok, posf, xf):
    mesh = plsc.VectorSubcoreMesh(core_axis_name="c", subcore_axis_name="s")

    @functools.partial(
        pl.kernel, mesh=mesh,
        out_type=jax.ShapeDtypeStruct((P, DIM), jnp.float32),
        scratch_types=[
            pltpu.VMEM((CH,), jnp.int32),
            pltpu.VMEM((CH,), jnp.int32),
            pltpu.VMEM((CH, DIM), jnp.float32),
            pltpu.SemaphoreType.DMA,
            pltpu.SemaphoreType.DMA,
        ],
    )
    def k(tok_hbm, pos_hbm, x_hbm, xs_hbm, tok_v, pos_v, rows_v, sem1, sem2):
        wid = lax.axis_index("s") * 2 + lax.axis_index("c")
        base = wid * (NA // NW)

        def body(c, carry):
            off = base + c * CH
            pltpu.sync_copy(tok_hbm.at[pl.ds(off, CH)], tok_v)
            pltpu.sync_copy(pos_hbm.at[pl.ds(off, CH)], pos_v)
            pltpu.async_copy(x_hbm.at[tok_v], rows_v, sem1).wait()
            pltpu.async_copy(rows_v, xs_hbm.at[pos_v], sem2).wait()
            return carry

        lax.fori_loop(0, NA // NW // CH, body, 0)

    return k(tok, posf, xf)


def _combine_sc(pos0, pos1, outs):
    mesh = plsc.VectorSubcoreMesh(core_axis_name="c", subcore_axis_name="s")

    @functools.partial(
        pl.kernel, mesh=mesh,
        out_type=(jax.ShapeDtypeStruct((NTOK, DIM), jnp.float32),
                  jax.ShapeDtypeStruct((NTOK, DIM), jnp.float32)),
        scratch_types=[
            pltpu.VMEM((CH,), jnp.int32),
            pltpu.VMEM((CH, DIM), jnp.float32),
            pltpu.SemaphoreType.DMA,
        ],
    )
    def k(p0_hbm, p1_hbm, outs_hbm, g0_hbm, g1_hbm, idx_v, rows_v, sem):
        wid = lax.axis_index("s") * 2 + lax.axis_index("c")
        base = wid * (NTOK // NW)

        def body(c, carry):
            off = base + c * CH
            pltpu.sync_copy(p0_hbm.at[pl.ds(off, CH)], idx_v)
            pltpu.async_copy(outs_hbm.at[idx_v], rows_v, sem).wait()
            pltpu.sync_copy(rows_v, g0_hbm.at[pl.ds(off, CH)])
            pltpu.sync_copy(p1_hbm.at[pl.ds(off, CH)], idx_v)
            pltpu.async_copy(outs_hbm.at[idx_v], rows_v, sem).wait()
            pltpu.sync_copy(rows_v, g1_hbm.at[pl.ds(off, CH)])
            return carry

        lax.fori_loop(0, NTOK // NW // CH, body, 0)

    return k(pos0, pos1, outs)


def _k1_body(be_ref, xs_ref, w1_ref, w3_ref, h_ref):
    del be_ref
    xb = xs_ref[...]
    t1 = lax.dot_general(xb, w1_ref[0], (((1,), (1,)), ((), ())),
                         preferred_element_type=jnp.float32)
    t3 = lax.dot_general(xb, w3_ref[0], (((1,), (1,)), ((), ())),
                         preferred_element_type=jnp.float32)
    h_ref[...] = ((t1 / (1.0 + jnp.exp(-t1))) * t3).astype(jnp.bfloat16)


def _run_k1(be, xs, w1, w3):
    gs = pltpu.PrefetchScalarGridSpec(
        num_scalar_prefetch=1,
        grid=(NB,),
        in_specs=[
            pl.BlockSpec((BT, DIM), lambda b, be: (b, 0)),
            pl.BlockSpec((1, INTER, DIM), lambda b, be: (be[b], 0, 0)),
            pl.BlockSpec((1, INTER, DIM), lambda b, be: (be[b], 0, 0)),
        ],
        out_specs=pl.BlockSpec((BT, INTER), lambda b, be: (b, 0)),
    )
    return pl.pallas_call(
        _k1_body, grid_spec=gs,
        out_shape=jax.ShapeDtypeStruct((P, INTER), jnp.bfloat16),
    )(be, xs, w1, w3)


def _k2_body(be_ref, h_ref, w2_ref, o_ref):
    del be_ref
    hb = h_ref[...].astype(jnp.float32)
    o_ref[...] = lax.dot_general(hb, w2_ref[0], (((1,), (1,)), ((), ())),
                                 preferred_element_type=jnp.float32)


def _run_k2(be, h, w2):
    gs = pltpu.PrefetchScalarGridSpec(
        num_scalar_prefetch=1,
        grid=(NB,),
        in_specs=[
            pl.BlockSpec((BT, INTER), lambda b, be: (b, 0)),
            pl.BlockSpec((1, DIM, INTER), lambda b, be: (be[b], 0, 0)),
        ],
        out_specs=pl.BlockSpec((BT, DIM), lambda b, be: (b, 0)),
    )
    return pl.pallas_call(
        _k2_body, grid_spec=gs,
        out_shape=jax.ShapeDtypeStruct((P, DIM), jnp.float32),
    )(be, h, w2)


SBT = 256


def _sk1_body(x_ref, sw1_ref, sw3_ref, hs_ref):
    xb = x_ref[...]
    t1 = lax.dot_general(xb, sw1_ref[0], (((1,), (1,)), ((), ())),
                         preferred_element_type=jnp.float32)
    t3 = lax.dot_general(xb, sw3_ref[0], (((1,), (1,)), ((), ())),
                         preferred_element_type=jnp.float32)
    hs_ref[0] = ((t1 / (1.0 + jnp.exp(-t1))) * t3).astype(jnp.bfloat16)


def _run_sk1(xf, sw1e, sw3e):
    return pl.pallas_call(
        _sk1_body,
        grid=(2, NTOK // SBT),
        in_specs=[
            pl.BlockSpec((SBT, DIM), lambda s, t: (t, 0)),
            pl.BlockSpec((1, INTER, DIM), lambda s, t: (s, 0, 0)),
            pl.BlockSpec((1, INTER, DIM), lambda s, t: (s, 0, 0)),
        ],
        out_specs=pl.BlockSpec((1, SBT, INTER), lambda s, t: (s, t, 0)),
        out_shape=jax.ShapeDtypeStruct((2, NTOK, INTER), jnp.bfloat16),
    )(xf, sw1e, sw3e)


def _sk2_body(hs_ref, sw2_ref, g0_ref, g1_ref, wt_ref, y_ref):
    s = pl.program_id(1)

    @pl.when(s == 0)
    def _():
        li = lax.broadcasted_iota(jnp.int32, (SBT, LANES), 1)
        w0 = jnp.sum(jnp.where(li == 0, wt_ref[...], 0.0), axis=1,
                     keepdims=True)
        w1 = jnp.sum(jnp.where(li == 1, wt_ref[...], 0.0), axis=1,
                     keepdims=True)
        y_ref[...] = w0 * g0_ref[...] + w1 * g1_ref[...]

    hb = hs_ref[0].astype(jnp.float32)
    y_ref[...] += lax.dot_general(hb, sw2_ref[...], (((1,), (1,)), ((), ())),
                                  preferred_element_type=jnp.float32)


def _run_sk2(hs, sw2, g0, g1, wt):
    return pl.pallas_call(
        _sk2_body,
        grid=(NTOK // SBT, 2),
        in_specs=[
            pl.BlockSpec((1, SBT, INTER), lambda t, s: (s, t, 0)),
            pl.BlockSpec((DIM, INTER), lambda t, s: (0, s)),
            pl.BlockSpec((SBT, DIM), lambda t, s: (t, 0)),
            pl.BlockSpec((SBT, DIM), lambda t, s: (t, 0)),
            pl.BlockSpec((SBT, LANES), lambda t, s: (t, 0)),
        ],
        out_specs=pl.BlockSpec((SBT, DIM), lambda t, s: (t, 0)),
        out_shape=jax.ShapeDtypeStruct((NTOK, DIM), jnp.float32),
    )(hs, sw2, g0, g1, wt)


def kernel(x, gate_w, w1, w2, w3, sw1, sw2, sw3):
    shape = x.shape
    xf = x.reshape(NTOK, DIM)
    gwp = jnp.zeros((LANES, DIM), jnp.float32).at[:NE].set(gate_w)

    wt, ints, sizesf = _run_gate(xf, gwp)

    sizes = sizesf[0, :NE].astype(jnp.int32)
    nblk = (sizes + BT - 1) // BT
    pad_off = jnp.concatenate(
        [jnp.zeros((1,), jnp.int32), jnp.cumsum(nblk) * BT])
    e0, e1 = ints[:, 0], ints[:, 1]
    pos0 = pad_off[e0] + ints[:, 2]
    pos1 = pad_off[e1] + ints[:, 3]
    posf = jnp.stack([pos0, pos1], axis=1).reshape(NA)
    tok = jnp.repeat(jnp.arange(NTOK, dtype=jnp.int32), 2)
    bb = jnp.arange(NB, dtype=jnp.int32) * BT
    be = jnp.clip(
        jnp.searchsorted(pad_off, bb, side="right").astype(jnp.int32) - 1,
        0, NE - 1)

    xs = _dispatch_sc(tok, posf, xf)
    h = _run_k1(be, xs, w1, w3)
    outs = _run_k2(be, h, w2)
    g0, g1 = _combine_sc(pos0, pos1, outs)

    sw1e = sw1.reshape(2, INTER, DIM)
    sw3e = sw3.reshape(2, INTER, DIM)
    hs = _run_sk1(xf, sw1e, sw3e)
    y = _run_sk2(hs, sw2, g0, g1, wt)
    return y.reshape(shape)


# speedup vs baseline: 2.1596x; 1.0273x over previous
import functools

import jax
import jax.numpy as jnp
from jax import lax
from jax.experimental import pallas as pl
from jax.experimental.pallas import tpu as pltpu
from jax.experimental.pallas import tpu_sc as plsc

DIM = 2048
INTER = 1408
NE = 16
NTOK = 4096
NA = 2 * NTOK
BT = 128
NB = NA // BT + NE
P = NB * BT
LANES = 128
GT = 1024
NW = 32
CH = 32


def _gate_body(x_ref, gw_ref, wt_ref, ints_ref, sizes_ref, carry_ref):
    t = pl.program_id(0)

    @pl.when(t == 0)
    def _():
        carry_ref[...] = jnp.zeros_like(carry_ref)

    xb = x_ref[...]
    logits = lax.dot_general(
        xb, gw_ref[...], (((1,), (1,)), ((), ())),
        preferred_element_type=jnp.float32)
    li = lax.broadcasted_iota(jnp.int32, (GT, LANES), 1)
    valid = li < NE
    logits = jnp.where(valid, logits, -1e30)
    m = jnp.max(logits, axis=1, keepdims=True)
    p = jnp.exp(logits - m)
    p = p / jnp.sum(p, axis=1, keepdims=True)
    v1 = jnp.max(p, axis=1, keepdims=True)
    e0 = jnp.min(jnp.where(p == v1, li, LANES), axis=1, keepdims=True)
    p2 = jnp.where(valid & (li != e0), p, -1.0)
    v2 = jnp.max(p2, axis=1, keepdims=True)
    e1 = jnp.min(jnp.where(p2 == v2, li, LANES), axis=1, keepdims=True)
    wt_ref[...] = jnp.where(li == 0, v1, jnp.where(li == 1, v2, 0.0))
    oh0 = (li == e0).astype(jnp.float32)
    oh1 = (li == e1).astype(jnp.float32)
    inc = oh0 + oh1
    ii = lax.broadcasted_iota(jnp.int32, (GT, GT), 0)
    jj = lax.broadcasted_iota(jnp.int32, (GT, GT), 1)
    ltri = (ii > jj).astype(jnp.float32)
    prefix = lax.dot_general(
        ltri, inc, (((1,), (0,)), ((), ())),
        preferred_element_type=jnp.float32)
    prefix = prefix + carry_ref[0:1, :]
    r0 = jnp.sum(jnp.where(li == e0, prefix, 0.0), axis=1, keepdims=True)
    r1 = jnp.sum(jnp.where(li == e1, prefix, 0.0), axis=1, keepdims=True)
    carry_ref[0:1, :] = carry_ref[0:1, :] + jnp.sum(inc, axis=0, keepdims=True)
    ints_ref[...] = jnp.where(
        li == 0, e0,
        jnp.where(li == 1, e1,
                  jnp.where(li == 2, r0.astype(jnp.int32),
                            jnp.where(li == 3, r1.astype(jnp.int32), 0))))

    @pl.when(t == NTOK // GT - 1)
    def _():
        sizes_ref[...] = carry_ref[...]


def _run_gate(xf, gwp):
    return pl.pallas_call(
        _gate_body,
        grid=(NTOK // GT,),
        in_specs=[
            pl.BlockSpec((GT, DIM), lambda t: (t, 0)),
            pl.BlockSpec((LANES, DIM), lambda t: (0, 0)),
        ],
        out_specs=[
            pl.BlockSpec((GT, LANES), lambda t: (t, 0)),
            pl.BlockSpec((GT, LANES), lambda t: (t, 0)),
            pl.BlockSpec((8, LANES), lambda t: (0, 0)),
        ],
        out_shape=[
            jax.ShapeDtypeStruct((NTOK, LANES), jnp.float32),
            jax.ShapeDtypeStruct((NTOK, LANES), jnp.int32),
            jax.ShapeDtypeStruct((8, LANES), jnp.float32),
        ],
        scratch_shapes=[pltpu.VMEM((8, LANES), jnp.float32)],
    )(xf, gwp)


def _dispatch_sc(pos0, pos1, xf):
    mesh = plsc.VectorSubcoreMesh(core_axis_name="c", subcore_axis_name="s")

    @functools.partial(
        pl.kernel, mesh=mesh,
        out_type=jax.ShapeDtypeStruct((P, DIM), jnp.float32),
        scratch_types=[
            pltpu.VMEM((CH,), jnp.int32),
            pltpu.VMEM((CH,), jnp.int32),
            pltpu.VMEM((CH, DIM), jnp.float32),
            pltpu.SemaphoreType.DMA,
            pltpu.SemaphoreType.DMA,
        ],
    )
    def k(p0_hbm, p1_hbm, x_hbm, xs_hbm, p0_v, p1_v, rows_v, sem1, sem2):
        wid = lax.axis_index("s") * 2 + lax.axis_index("c")
        base = wid * (NTOK // NW)

        def body(c, carry):
            off = base + c * CH
            pltpu.sync_copy(p0_hbm.at[pl.ds(off, CH)], p0_v)
            pltpu.sync_copy(p1_hbm.at[pl.ds(off, CH)], p1_v)
            pltpu.sync_copy(x_hbm.at[pl.ds(off, CH)], rows_v)
            c1 = pltpu.async_copy(rows_v, xs_hbm.at[p0_v], sem1)
            c2 = pltpu.async_copy(rows_v, xs_hbm.at[p1_v], sem2)
            c1.wait()
            c2.wait()
            return carry

        lax.fori_loop(0, NTOK // NW // CH, body, 0)

    return k(pos0, pos1, xf)


def _combine_sc(pos0, pos1, outs):
    mesh = plsc.VectorSubcoreMesh(core_axis_name="c", subcore_axis_name="s")

    @functools.partial(
        pl.kernel, mesh=mesh,
        out_type=(jax.ShapeDtypeStruct((NTOK, DIM), jnp.float32),
                  jax.ShapeDtypeStruct((NTOK, DIM), jnp.float32)),
        scratch_types=[
            pltpu.VMEM((CH,), jnp.int32),
            pltpu.VMEM((CH, DIM), jnp.float32),
            pltpu.SemaphoreType.DMA,
        ],
    )
    def k(p0_hbm, p1_hbm, outs_hbm, g0_hbm, g1_hbm, idx_v, rows_v, sem):
        wid = lax.axis_index("s") * 2 + lax.axis_index("c")
        base = wid * (NTOK // NW)

        def body(c, carry):
            off = base + c * CH
            pltpu.sync_copy(p0_hbm.at[pl.ds(off, CH)], idx_v)
            pltpu.async_copy(outs_hbm.at[idx_v], rows_v, sem).wait()
            pltpu.sync_copy(rows_v, g0_hbm.at[pl.ds(off, CH)])
            pltpu.sync_copy(p1_hbm.at[pl.ds(off, CH)], idx_v)
            pltpu.async_copy(outs_hbm.at[idx_v], rows_v, sem).wait()
            pltpu.sync_copy(rows_v, g1_hbm.at[pl.ds(off, CH)])
            return carry

        lax.fori_loop(0, NTOK // NW // CH, body, 0)

    return k(pos0, pos1, outs)


def _k1_body(be_ref, xs_ref, w1_ref, w3_ref, h_ref):
    del be_ref
    xb = xs_ref[...]
    t1 = lax.dot_general(xb, w1_ref[0], (((1,), (1,)), ((), ())),
                         preferred_element_type=jnp.float32)
    t3 = lax.dot_general(xb, w3_ref[0], (((1,), (1,)), ((), ())),
                         preferred_element_type=jnp.float32)
    h_ref[...] = ((t1 / (1.0 + jnp.exp(-t1))) * t3).astype(jnp.bfloat16)


def _run_k1(be, xs, w1, w3):
    gs = pltpu.PrefetchScalarGridSpec(
        num_scalar_prefetch=1,
        grid=(NB,),
        in_specs=[
            pl.BlockSpec((BT, DIM), lambda b, be: (b, 0)),
            pl.BlockSpec((1, INTER, DIM), lambda b, be: (be[b], 0, 0)),
            pl.BlockSpec((1, INTER, DIM), lambda b, be: (be[b], 0, 0)),
        ],
        out_specs=pl.BlockSpec((BT, INTER), lambda b, be: (b, 0)),
    )
    return pl.pallas_call(
        _k1_body, grid_spec=gs,
        out_shape=jax.ShapeDtypeStruct((P, INTER), jnp.bfloat16),
    )(be, xs, w1, w3)


def _k2_body(be_ref, h_ref, w2_ref, o_ref):
    del be_ref
    hb = h_ref[...].astype(jnp.float32)
    o_ref[...] = lax.dot_general(hb, w2_ref[0], (((1,), (1,)), ((), ())),
                                 preferred_element_type=jnp.float32)


def _run_k2(be, h, w2):
    gs = pltpu.PrefetchScalarGridSpec(
        num_scalar_prefetch=1,
        grid=(NB,),
        in_specs=[
            pl.BlockSpec((BT, INTER), lambda b, be: (b, 0)),
            pl.BlockSpec((1, DIM, INTER), lambda b, be: (be[b], 0, 0)),
        ],
        out_specs=pl.BlockSpec((BT, DIM), lambda b, be: (b, 0)),
    )
    return pl.pallas_call(
        _k2_body, grid_spec=gs,
        out_shape=jax.ShapeDtypeStruct((P, DIM), jnp.float32),
    )(be, h, w2)


SBT = 256


def _sk1_body(x_ref, sw1_ref, sw3_ref, hs_ref):
    xb = x_ref[...]
    t1 = lax.dot_general(xb, sw1_ref[0], (((1,), (1,)), ((), ())),
                         preferred_element_type=jnp.float32)
    t3 = lax.dot_general(xb, sw3_ref[0], (((1,), (1,)), ((), ())),
                         preferred_element_type=jnp.float32)
    hs_ref[0] = ((t1 / (1.0 + jnp.exp(-t1))) * t3).astype(jnp.bfloat16)


def _run_sk1(xf, sw1e, sw3e):
    return pl.pallas_call(
        _sk1_body,
        grid=(2, NTOK // SBT),
        in_specs=[
            pl.BlockSpec((SBT, DIM), lambda s, t: (t, 0)),
            pl.BlockSpec((1, INTER, DIM), lambda s, t: (s, 0, 0)),
            pl.BlockSpec((1, INTER, DIM), lambda s, t: (s, 0, 0)),
        ],
        out_specs=pl.BlockSpec((1, SBT, INTER), lambda s, t: (s, t, 0)),
        out_shape=jax.ShapeDtypeStruct((2, NTOK, INTER), jnp.bfloat16),
    )(xf, sw1e, sw3e)


def _sk2_body(hs_ref, sw2_ref, g0_ref, g1_ref, wt_ref, y_ref):
    s = pl.program_id(1)

    @pl.when(s == 0)
    def _():
        li = lax.broadcasted_iota(jnp.int32, (SBT, LANES), 1)
        w0 = jnp.sum(jnp.where(li == 0, wt_ref[...], 0.0), axis=1,
                     keepdims=True)
        w1 = jnp.sum(jnp.where(li == 1, wt_ref[...], 0.0), axis=1,
                     keepdims=True)
        y_ref[...] = w0 * g0_ref[...] + w1 * g1_ref[...]

    hb = hs_ref[0].astype(jnp.float32)
    y_ref[...] += lax.dot_general(hb, sw2_ref[...], (((1,), (1,)), ((), ())),
                                  preferred_element_type=jnp.float32)


def _run_sk2(hs, sw2, g0, g1, wt):
    return pl.pallas_call(
        _sk2_body,
        grid=(NTOK // SBT, 2),
        in_specs=[
            pl.BlockSpec((1, SBT, INTER), lambda t, s: (s, t, 0)),
            pl.BlockSpec((DIM, INTER), lambda t, s: (0, s)),
            pl.BlockSpec((SBT, DIM), lambda t, s: (t, 0)),
            pl.BlockSpec((SBT, DIM), lambda t, s: (t, 0)),
            pl.BlockSpec((SBT, LANES), lambda t, s: (t, 0)),
        ],
        out_specs=pl.BlockSpec((SBT, DIM), lambda t, s: (t, 0)),
        out_shape=jax.ShapeDtypeStruct((NTOK, DIM), jnp.float32),
    )(hs, sw2, g0, g1, wt)


def kernel(x, gate_w, w1, w2, w3, sw1, sw2, sw3):
    shape = x.shape
    xf = x.reshape(NTOK, DIM)
    gwp = jnp.zeros((LANES, DIM), jnp.float32).at[:NE].set(gate_w)

    wt, ints, sizesf = _run_gate(xf, gwp)

    sizes = sizesf[0, :NE].astype(jnp.int32)
    nblk = (sizes + BT - 1) // BT
    pad_off = jnp.concatenate(
        [jnp.zeros((1,), jnp.int32), jnp.cumsum(nblk) * BT])
    e0, e1 = ints[:, 0], ints[:, 1]
    pos0 = pad_off[e0] + ints[:, 2]
    pos1 = pad_off[e1] + ints[:, 3]
    bb = jnp.arange(NB, dtype=jnp.int32) * BT
    be = jnp.clip(
        jnp.searchsorted(pad_off, bb, side="right").astype(jnp.int32) - 1,
        0, NE - 1)

    xs = _dispatch_sc(pos0, pos1, xf)
    sw1e = sw1.reshape(2, INTER, DIM)
    sw3e = sw3.reshape(2, INTER, DIM)
    hs = _run_sk1(xf, sw1e, sw3e)
    h = _run_k1(be, xs, w1, w3)
    outs = _run_k2(be, h, w2)
    g0, g1 = _combine_sc(pos0, pos1, outs)
    y = _run_sk2(hs, sw2, g0, g1, wt)
    return y.reshape(shape)


# speedup vs baseline: 2.6773x; 1.2397x over previous
import functools

import jax
import jax.numpy as jnp
from jax import lax
from jax.experimental import pallas as pl
from jax.experimental.pallas import tpu as pltpu
from jax.experimental.pallas import tpu_sc as plsc

DIM = 2048
INTER = 1408
NE = 16
NTOK = 4096
NA = 2 * NTOK
BT = 256
NB = NA // BT + NE
P = NB * BT
LANES = 128
GT = 1024
NW = 32
CH = 32


def _gate_body(x_ref, gw_ref, wt_ref, ints_ref, sizes_ref, carry_ref):
    t = pl.program_id(0)

    @pl.when(t == 0)
    def _():
        carry_ref[...] = jnp.zeros_like(carry_ref)

    xb = x_ref[...]
    logits = lax.dot_general(
        xb, gw_ref[...], (((1,), (1,)), ((), ())),
        preferred_element_type=jnp.float32)
    li = lax.broadcasted_iota(jnp.int32, (GT, LANES), 1)
    valid = li < NE
    logits = jnp.where(valid, logits, -1e30)
    m = jnp.max(logits, axis=1, keepdims=True)
    p = jnp.exp(logits - m)
    p = p / jnp.sum(p, axis=1, keepdims=True)
    v1 = jnp.max(p, axis=1, keepdims=True)
    e0 = jnp.min(jnp.where(p == v1, li, LANES), axis=1, keepdims=True)
    p2 = jnp.where(valid & (li != e0), p, -1.0)
    v2 = jnp.max(p2, axis=1, keepdims=True)
    e1 = jnp.min(jnp.where(p2 == v2, li, LANES), axis=1, keepdims=True)
    wt_ref[...] = jnp.where(li == 0, v1, jnp.where(li == 1, v2, 0.0))
    oh0 = (li == e0).astype(jnp.float32)
    oh1 = (li == e1).astype(jnp.float32)
    inc = oh0 + oh1
    ii = lax.broadcasted_iota(jnp.int32, (GT, GT), 0)
    jj = lax.broadcasted_iota(jnp.int32, (GT, GT), 1)
    ltri = (ii > jj).astype(jnp.float32)
    prefix = lax.dot_general(
        ltri, inc, (((1,), (0,)), ((), ())),
        preferred_element_type=jnp.float32)
    prefix = prefix + carry_ref[0:1, :]
    r0 = jnp.sum(jnp.where(li == e0, prefix, 0.0), axis=1, keepdims=True)
    r1 = jnp.sum(jnp.where(li == e1, prefix, 0.0), axis=1, keepdims=True)
    carry_ref[0:1, :] = carry_ref[0:1, :] + jnp.sum(inc, axis=0, keepdims=True)
    ints_ref[...] = jnp.where(
        li == 0, e0,
        jnp.where(li == 1, e1,
                  jnp.where(li == 2, r0.astype(jnp.int32),
                            jnp.where(li == 3, r1.astype(jnp.int32), 0))))

    @pl.when(t == NTOK // GT - 1)
    def _():
        sizes_ref[...] = carry_ref[...]


def _run_gate(xf, gwp):
    return pl.pallas_call(
        _gate_body,
        grid=(NTOK // GT,),
        in_specs=[
            pl.BlockSpec((GT, DIM), lambda t: (t, 0)),
            pl.BlockSpec((LANES, DIM), lambda t: (0, 0)),
        ],
        out_specs=[
            pl.BlockSpec((GT, LANES), lambda t: (t, 0)),
            pl.BlockSpec((GT, LANES), lambda t: (t, 0)),
            pl.BlockSpec((8, LANES), lambda t: (0, 0)),
        ],
        out_shape=[
            jax.ShapeDtypeStruct((NTOK, LANES), jnp.float32),
            jax.ShapeDtypeStruct((NTOK, LANES), jnp.int32),
            jax.ShapeDtypeStruct((8, LANES), jnp.float32),
        ],
        scratch_shapes=[pltpu.VMEM((8, LANES), jnp.float32)],
    )(xf, gwp)


def _dispatch_sc(pos0, pos1, xf):
    mesh = plsc.VectorSubcoreMesh(core_axis_name="c", subcore_axis_name="s")

    @functools.partial(
        pl.kernel, mesh=mesh,
        out_type=jax.ShapeDtypeStruct((P, DIM), jnp.float32),
        scratch_types=[
            pltpu.VMEM((CH,), jnp.int32),
            pltpu.VMEM((CH,), jnp.int32),
            pltpu.VMEM((CH, DIM), jnp.float32),
            pltpu.SemaphoreType.DMA,
            pltpu.SemaphoreType.DMA,
        ],
    )
    def k(p0_hbm, p1_hbm, x_hbm, xs_hbm, p0_v, p1_v, rows_v, sem1, sem2):
        wid = lax.axis_index("s") * 2 + lax.axis_index("c")
        base = wid * (NTOK // NW)

        def body(c, carry):
            off = base + c * CH
            pltpu.sync_copy(p0_hbm.at[pl.ds(off, CH)], p0_v)
            pltpu.sync_copy(p1_hbm.at[pl.ds(off, CH)], p1_v)
            pltpu.sync_copy(x_hbm.at[pl.ds(off, CH)], rows_v)
            c1 = pltpu.async_copy(rows_v, xs_hbm.at[p0_v], sem1)
            c2 = pltpu.async_copy(rows_v, xs_hbm.at[p1_v], sem2)
            c1.wait()
            c2.wait()
            return carry

        lax.fori_loop(0, NTOK // NW // CH, body, 0)

    return k(pos0, pos1, xf)


def _combine_sc(pos0, pos1, outs):
    mesh = plsc.VectorSubcoreMesh(core_axis_name="c", subcore_axis_name="s")

    @functools.partial(
        pl.kernel, mesh=mesh,
        out_type=(jax.ShapeDtypeStruct((NTOK, DIM), jnp.float32),
                  jax.ShapeDtypeStruct((NTOK, DIM), jnp.float32)),
        scratch_types=[
            pltpu.VMEM((CH,), jnp.int32),
            pltpu.VMEM((CH, DIM), jnp.float32),
            pltpu.SemaphoreType.DMA,
        ],
    )
    def k(p0_hbm, p1_hbm, outs_hbm, g0_hbm, g1_hbm, idx_v, rows_v, sem):
        wid = lax.axis_index("s") * 2 + lax.axis_index("c")
        base = wid * (NTOK // NW)

        def body(c, carry):
            off = base + c * CH
            pltpu.sync_copy(p0_hbm.at[pl.ds(off, CH)], idx_v)
            pltpu.async_copy(outs_hbm.at[idx_v], rows_v, sem).wait()
            pltpu.sync_copy(rows_v, g0_hbm.at[pl.ds(off, CH)])
            pltpu.sync_copy(p1_hbm.at[pl.ds(off, CH)], idx_v)
            pltpu.async_copy(outs_hbm.at[idx_v], rows_v, sem).wait()
            pltpu.sync_copy(rows_v, g1_hbm.at[pl.ds(off, CH)])
            return carry

        lax.fori_loop(0, NTOK // NW // CH, body, 0)

    return k(pos0, pos1, outs)


def _k1_body(be_ref, xs_ref, w1_ref, w3_ref, h_ref):
    del be_ref
    xb = xs_ref[...]
    t1 = lax.dot_general(xb, w1_ref[0], (((1,), (1,)), ((), ())),
                         preferred_element_type=jnp.float32)
    t3 = lax.dot_general(xb, w3_ref[0], (((1,), (1,)), ((), ())),
                         preferred_element_type=jnp.float32)
    h_ref[...] = ((t1 / (1.0 + jnp.exp(-t1))) * t3).astype(jnp.bfloat16)


def _run_k1(be, xs, w1, w3):
    gs = pltpu.PrefetchScalarGridSpec(
        num_scalar_prefetch=1,
        grid=(NB,),
        in_specs=[
            pl.BlockSpec((BT, DIM), lambda b, be: (b, 0)),
            pl.BlockSpec((1, INTER, DIM), lambda b, be: (be[b], 0, 0)),
            pl.BlockSpec((1, INTER, DIM), lambda b, be: (be[b], 0, 0)),
        ],
        out_specs=pl.BlockSpec((BT, INTER), lambda b, be: (b, 0)),
    )
    return pl.pallas_call(
        _k1_body, grid_spec=gs,
        out_shape=jax.ShapeDtypeStruct((P, INTER), jnp.bfloat16),
    )(be, xs, w1, w3)


def _k2_body(be_ref, h_ref, w2_ref, o_ref):
    del be_ref
    hb = h_ref[...].astype(jnp.float32)
    o_ref[...] = lax.dot_general(hb, w2_ref[0], (((1,), (1,)), ((), ())),
                                 preferred_element_type=jnp.float32)


def _run_k2(be, h, w2):
    gs = pltpu.PrefetchScalarGridSpec(
        num_scalar_prefetch=1,
        grid=(NB,),
        in_specs=[
            pl.BlockSpec((BT, INTER), lambda b, be: (b, 0)),
            pl.BlockSpec((1, DIM, INTER), lambda b, be: (be[b], 0, 0)),
        ],
        out_specs=pl.BlockSpec((BT, DIM), lambda b, be: (b, 0)),
    )
    return pl.pallas_call(
        _k2_body, grid_spec=gs,
        out_shape=jax.ShapeDtypeStruct((P, DIM), jnp.float32),
    )(be, h, w2)


SBT = 256


def _sk1_body(x_ref, sw1_ref, sw3_ref, hs_ref):
    xb = x_ref[...]
    t1 = lax.dot_general(xb, sw1_ref[0], (((1,), (1,)), ((), ())),
                         preferred_element_type=jnp.float32)
    t3 = lax.dot_general(xb, sw3_ref[0], (((1,), (1,)), ((), ())),
                         preferred_element_type=jnp.float32)
    hs_ref[0] = ((t1 / (1.0 + jnp.exp(-t1))) * t3).astype(jnp.bfloat16)


def _run_sk1(xf, sw1e, sw3e):
    return pl.pallas_call(
        _sk1_body,
        grid=(2, NTOK // SBT),
        in_specs=[
            pl.BlockSpec((SBT, DIM), lambda s, t: (t, 0)),
            pl.BlockSpec((1, INTER, DIM), lambda s, t: (s, 0, 0)),
            pl.BlockSpec((1, INTER, DIM), lambda s, t: (s, 0, 0)),
        ],
        out_specs=pl.BlockSpec((1, SBT, INTER), lambda s, t: (s, t, 0)),
        out_shape=jax.ShapeDtypeStruct((2, NTOK, INTER), jnp.bfloat16),
    )(xf, sw1e, sw3e)


def _sk2_body(hs_ref, sw2_ref, g0_ref, g1_ref, wt_ref, y_ref):
    s = pl.program_id(1)

    @pl.when(s == 0)
    def _():
        li = lax.broadcasted_iota(jnp.int32, (SBT, LANES), 1)
        w0 = jnp.sum(jnp.where(li == 0, wt_ref[...], 0.0), axis=1,
                     keepdims=True)
        w1 = jnp.sum(jnp.where(li == 1, wt_ref[...], 0.0), axis=1,
                     keepdims=True)
        y_ref[...] = w0 * g0_ref[...] + w1 * g1_ref[...]

    hb = hs_ref[0].astype(jnp.float32)
    y_ref[...] += lax.dot_general(hb, sw2_ref[...], (((1,), (1,)), ((), ())),
                                  preferred_element_type=jnp.float32)


def _run_sk2(hs, sw2, g0, g1, wt):
    return pl.pallas_call(
        _sk2_body,
        grid=(NTOK // SBT, 2),
        in_specs=[
            pl.BlockSpec((1, SBT, INTER), lambda t, s: (s, t, 0)),
            pl.BlockSpec((DIM, INTER), lambda t, s: (0, s)),
            pl.BlockSpec((SBT, DIM), lambda t, s: (t, 0)),
            pl.BlockSpec((SBT, DIM), lambda t, s: (t, 0)),
            pl.BlockSpec((SBT, LANES), lambda t, s: (t, 0)),
        ],
        out_specs=pl.BlockSpec((SBT, DIM), lambda t, s: (t, 0)),
        out_shape=jax.ShapeDtypeStruct((NTOK, DIM), jnp.float32),
    )(hs, sw2, g0, g1, wt)


def kernel(x, gate_w, w1, w2, w3, sw1, sw2, sw3):
    shape = x.shape
    xf = x.reshape(NTOK, DIM)
    gwp = jnp.zeros((LANES, DIM), jnp.float32).at[:NE].set(gate_w)

    wt, ints, sizesf = _run_gate(xf, gwp)

    sizes = sizesf[0, :NE].astype(jnp.int32)
    nblk = (sizes + BT - 1) // BT
    pad_off = jnp.concatenate(
        [jnp.zeros((1,), jnp.int32), jnp.cumsum(nblk) * BT])
    e0, e1 = ints[:, 0], ints[:, 1]
    pos0 = pad_off[e0] + ints[:, 2]
    pos1 = pad_off[e1] + ints[:, 3]
    bb = jnp.arange(NB, dtype=jnp.int32) * BT
    be = jnp.clip(
        jnp.searchsorted(pad_off, bb, side="right").astype(jnp.int32) - 1,
        0, NE - 1)

    xs = _dispatch_sc(pos0, pos1, xf)
    sw1e = sw1.reshape(2, INTER, DIM)
    sw3e = sw3.reshape(2, INTER, DIM)
    hs = _run_sk1(xf, sw1e, sw3e)
    h = _run_k1(be, xs, w1, w3)
    outs = _run_k2(be, h, w2)
    g0, g1 = _combine_sc(pos0, pos1, outs)
    y = _run_sk2(hs, sw2, g0, g1, wt)
    return y.reshape(shape)


# speedup vs baseline: 2.9402x; 1.0982x over previous
import functools

import jax
import jax.numpy as jnp
from jax import lax
from jax.experimental import pallas as pl
from jax.experimental.pallas import tpu as pltpu
from jax.experimental.pallas import tpu_sc as plsc

DIM = 2048
INTER = 1408
NE = 16
NTOK = 4096
NA = 2 * NTOK
BT = 256
NB = NA // BT + NE
P = NB * BT
LANES = 128
GT = 1024
NW = 32
CH = 32


def _gate_body(x_ref, gw_ref, wt_ref, ints_ref, sizes_ref, carry_ref):
    t = pl.program_id(0)

    @pl.when(t == 0)
    def _():
        carry_ref[...] = jnp.zeros_like(carry_ref)

    xb = x_ref[...]
    logits = lax.dot_general(
        xb, gw_ref[...], (((1,), (1,)), ((), ())),
        preferred_element_type=jnp.float32)
    li = lax.broadcasted_iota(jnp.int32, (GT, LANES), 1)
    valid = li < NE
    logits = jnp.where(valid, logits, -1e30)
    m = jnp.max(logits, axis=1, keepdims=True)
    p = jnp.exp(logits - m)
    p = p / jnp.sum(p, axis=1, keepdims=True)
    v1 = jnp.max(p, axis=1, keepdims=True)
    e0 = jnp.min(jnp.where(p == v1, li, LANES), axis=1, keepdims=True)
    p2 = jnp.where(valid & (li != e0), p, -1.0)
    v2 = jnp.max(p2, axis=1, keepdims=True)
    e1 = jnp.min(jnp.where(p2 == v2, li, LANES), axis=1, keepdims=True)
    wt_ref[...] = jnp.where(li == 0, v1, jnp.where(li == 1, v2, 0.0))
    oh0 = (li == e0).astype(jnp.float32)
    oh1 = (li == e1).astype(jnp.float32)
    inc = oh0 + oh1
    ii = lax.broadcasted_iota(jnp.int32, (GT, GT), 0)
    jj = lax.broadcasted_iota(jnp.int32, (GT, GT), 1)
    ltri = (ii > jj).astype(jnp.float32)
    prefix = lax.dot_general(
        ltri, inc, (((1,), (0,)), ((), ())),
        preferred_element_type=jnp.float32)
    prefix = prefix + carry_ref[0:1, :]
    r0 = jnp.sum(jnp.where(li == e0, prefix, 0.0), axis=1, keepdims=True)
    r1 = jnp.sum(jnp.where(li == e1, prefix, 0.0), axis=1, keepdims=True)
    carry_ref[0:1, :] = carry_ref[0:1, :] + jnp.sum(inc, axis=0, keepdims=True)
    ints_ref[...] = jnp.where(
        li == 0, e0,
        jnp.where(li == 1, e1,
                  jnp.where(li == 2, r0.astype(jnp.int32),
                            jnp.where(li == 3, r1.astype(jnp.int32), 0))))

    @pl.when(t == NTOK // GT - 1)
    def _():
        sizes_ref[...] = carry_ref[...]


def _run_gate(xf, gwp):
    return pl.pallas_call(
        _gate_body,
        grid=(NTOK // GT,),
        in_specs=[
            pl.BlockSpec((GT, DIM), lambda t: (t, 0)),
            pl.BlockSpec((LANES, DIM), lambda t: (0, 0)),
        ],
        out_specs=[
            pl.BlockSpec((GT, LANES), lambda t: (t, 0)),
            pl.BlockSpec((GT, LANES), lambda t: (t, 0)),
            pl.BlockSpec((8, LANES), lambda t: (0, 0)),
        ],
        out_shape=[
            jax.ShapeDtypeStruct((NTOK, LANES), jnp.float32),
            jax.ShapeDtypeStruct((NTOK, LANES), jnp.int32),
            jax.ShapeDtypeStruct((8, LANES), jnp.float32),
        ],
        scratch_shapes=[pltpu.VMEM((8, LANES), jnp.float32)],
    )(xf, gwp)


def _dispatch_sc(pos0, pos1, xf):
    mesh = plsc.VectorSubcoreMesh(core_axis_name="c", subcore_axis_name="s")

    @functools.partial(
        pl.kernel, mesh=mesh,
        out_type=jax.ShapeDtypeStruct((P, DIM), jnp.float32),
        scratch_types=[
            pltpu.VMEM((CH,), jnp.int32),
            pltpu.VMEM((CH,), jnp.int32),
            pltpu.VMEM((CH, DIM), jnp.float32),
            pltpu.SemaphoreType.DMA,
            pltpu.SemaphoreType.DMA,
        ],
    )
    def k(p0_hbm, p1_hbm, x_hbm, xs_hbm, p0_v, p1_v, rows_v, sem1, sem2):
        wid = lax.axis_index("s") * 2 + lax.axis_index("c")
        base = wid * (NTOK // NW)

        def body(c, carry):
            off = base + c * CH
            pltpu.sync_copy(p0_hbm.at[pl.ds(off, CH)], p0_v)
            pltpu.sync_copy(p1_hbm.at[pl.ds(off, CH)], p1_v)
            pltpu.sync_copy(x_hbm.at[pl.ds(off, CH)], rows_v)
            c1 = pltpu.async_copy(rows_v, xs_hbm.at[p0_v], sem1)
            c2 = pltpu.async_copy(rows_v, xs_hbm.at[p1_v], sem2)
            c1.wait()
            c2.wait()
            return carry

        lax.fori_loop(0, NTOK // NW // CH, body, 0)

    return k(pos0, pos1, xf)


def _combine_sc(pos0, pos1, outs):
    mesh = plsc.VectorSubcoreMesh(core_axis_name="c", subcore_axis_name="s")

    @functools.partial(
        pl.kernel, mesh=mesh,
        out_type=(jax.ShapeDtypeStruct((NTOK, DIM), jnp.float32),
                  jax.ShapeDtypeStruct((NTOK, DIM), jnp.float32)),
        scratch_types=[
            pltpu.VMEM((CH,), jnp.int32),
            pltpu.VMEM((CH, DIM), jnp.float32),
            pltpu.SemaphoreType.DMA,
        ],
    )
    def k(p0_hbm, p1_hbm, outs_hbm, g0_hbm, g1_hbm, idx_v, rows_v, sem):
        wid = lax.axis_index("s") * 2 + lax.axis_index("c")
        base = wid * (NTOK // NW)

        def body(c, carry):
            off = base + c * CH
            pltpu.sync_copy(p0_hbm.at[pl.ds(off, CH)], idx_v)
            pltpu.async_copy(outs_hbm.at[idx_v], rows_v, sem).wait()
            pltpu.sync_copy(rows_v, g0_hbm.at[pl.ds(off, CH)])
            pltpu.sync_copy(p1_hbm.at[pl.ds(off, CH)], idx_v)
            pltpu.async_copy(outs_hbm.at[idx_v], rows_v, sem).wait()
            pltpu.sync_copy(rows_v, g1_hbm.at[pl.ds(off, CH)])
            return carry

        lax.fori_loop(0, NTOK // NW // CH, body, 0)

    return k(pos0, pos1, outs)


def _k1_body(be_ref, xs_ref, w1_ref, w3_ref, h_ref):
    del be_ref
    xb = xs_ref[...]
    t1 = lax.dot_general(xb, w1_ref[0], (((1,), (1,)), ((), ())),
                         preferred_element_type=jnp.float32)
    t3 = lax.dot_general(xb, w3_ref[0], (((1,), (1,)), ((), ())),
                         preferred_element_type=jnp.float32)
    h_ref[...] = ((t1 / (1.0 + jnp.exp(-t1))) * t3).astype(jnp.bfloat16)


def _run_k1(be, xs, w1, w3):
    gs = pltpu.PrefetchScalarGridSpec(
        num_scalar_prefetch=1,
        grid=(NB,),
        in_specs=[
            pl.BlockSpec((BT, DIM), lambda b, be: (b, 0)),
            pl.BlockSpec((1, INTER, DIM), lambda b, be: (be[b], 0, 0)),
            pl.BlockSpec((1, INTER, DIM), lambda b, be: (be[b], 0, 0)),
        ],
        out_specs=pl.BlockSpec((BT, INTER), lambda b, be: (b, 0)),
    )
    return pl.pallas_call(
        _k1_body, grid_spec=gs,
        out_shape=jax.ShapeDtypeStruct((P, INTER), jnp.bfloat16),
    )(be, xs, w1, w3)


def _k2_body(be_ref, h_ref, w2_ref, o_ref):
    del be_ref
    hb = h_ref[...].astype(jnp.float32)
    o_ref[...] = lax.dot_general(hb, w2_ref[0], (((1,), (1,)), ((), ())),
                                 preferred_element_type=jnp.float32)


def _run_k2(be, h, w2):
    gs = pltpu.PrefetchScalarGridSpec(
        num_scalar_prefetch=1,
        grid=(NB,),
        in_specs=[
            pl.BlockSpec((BT, INTER), lambda b, be: (b, 0)),
            pl.BlockSpec((1, DIM, INTER), lambda b, be: (be[b], 0, 0)),
        ],
        out_specs=pl.BlockSpec((BT, DIM), lambda b, be: (b, 0)),
    )
    return pl.pallas_call(
        _k2_body, grid_spec=gs,
        out_shape=jax.ShapeDtypeStruct((P, DIM), jnp.float32),
    )(be, h, w2)


SBT = 256


def _sk1_body(x_ref, sw1_ref, sw3_ref, *rest):
    hs_ref = rest[-1]
    xb = x_ref[...]
    t1 = lax.dot_general(xb, sw1_ref[0], (((1,), (1,)), ((), ())),
                         preferred_element_type=jnp.float32)
    t3 = lax.dot_general(xb, sw3_ref[0], (((1,), (1,)), ((), ())),
                         preferred_element_type=jnp.float32)
    hs_ref[...] = ((t1 / (1.0 + jnp.exp(-t1))) * t3).astype(jnp.bfloat16)


def _run_sk1(xf, sw1e, sw3e, s, dep=None):
    deps = [] if dep is None else [dep]
    dep_specs = [] if dep is None else [
        pl.BlockSpec((8, LANES), lambda t: (0, 0))]
    return pl.pallas_call(
        _sk1_body,
        grid=(NTOK // SBT,),
        in_specs=[
            pl.BlockSpec((SBT, DIM), lambda t: (t, 0)),
            pl.BlockSpec((1, INTER, DIM), lambda t: (s, 0, 0)),
            pl.BlockSpec((1, INTER, DIM), lambda t: (s, 0, 0)),
        ] + dep_specs,
        out_specs=pl.BlockSpec((SBT, INTER), lambda t: (t, 0)),
        out_shape=jax.ShapeDtypeStruct((NTOK, INTER), jnp.bfloat16),
    )(xf, sw1e, sw3e, *deps)


def _sk2_body(hs0_ref, hs1_ref, sw2a_ref, sw2b_ref, g0_ref, g1_ref,
              wt_ref, y_ref):
    li = lax.broadcasted_iota(jnp.int32, (SBT, LANES), 1)
    w0 = jnp.sum(jnp.where(li == 0, wt_ref[...], 0.0), axis=1, keepdims=True)
    w1 = jnp.sum(jnp.where(li == 1, wt_ref[...], 0.0), axis=1, keepdims=True)
    acc = w0 * g0_ref[...] + w1 * g1_ref[...]
    acc += lax.dot_general(hs0_ref[...].astype(jnp.float32), sw2a_ref[...],
                           (((1,), (1,)), ((), ())),
                           preferred_element_type=jnp.float32)
    acc += lax.dot_general(hs1_ref[...].astype(jnp.float32), sw2b_ref[...],
                           (((1,), (1,)), ((), ())),
                           preferred_element_type=jnp.float32)
    y_ref[...] = acc


def _run_sk2(hs0, hs1, sw2, g0, g1, wt):
    return pl.pallas_call(
        _sk2_body,
        grid=(NTOK // SBT,),
        in_specs=[
            pl.BlockSpec((SBT, INTER), lambda t: (t, 0)),
            pl.BlockSpec((SBT, INTER), lambda t: (t, 0)),
            pl.BlockSpec((DIM, INTER), lambda t: (0, 0)),
            pl.BlockSpec((DIM, INTER), lambda t: (0, 1)),
            pl.BlockSpec((SBT, DIM), lambda t: (t, 0)),
            pl.BlockSpec((SBT, DIM), lambda t: (t, 0)),
            pl.BlockSpec((SBT, LANES), lambda t: (t, 0)),
        ],
        out_specs=pl.BlockSpec((SBT, DIM), lambda t: (t, 0)),
        out_shape=jax.ShapeDtypeStruct((NTOK, DIM), jnp.float32),
    )(hs0, hs1, sw2, sw2, g0, g1, wt)


def kernel(x, gate_w, w1, w2, w3, sw1, sw2, sw3):
    shape = x.shape
    xf = x.reshape(NTOK, DIM)
    gwp = jnp.zeros((LANES, DIM), jnp.float32).at[:NE].set(gate_w)

    wt, ints, sizesf = _run_gate(xf, gwp)

    sizes = sizesf[0, :NE].astype(jnp.int32)
    nblk = (sizes + BT - 1) // BT
    pad_off = jnp.concatenate(
        [jnp.zeros((1,), jnp.int32), jnp.cumsum(nblk) * BT])
    e0, e1 = ints[:, 0], ints[:, 1]
    pos0 = pad_off[e0] + ints[:, 2]
    pos1 = pad_off[e1] + ints[:, 3]
    bb = jnp.arange(NB, dtype=jnp.int32) * BT
    be = jnp.clip(
        jnp.searchsorted(pad_off, bb, side="right").astype(jnp.int32) - 1,
        0, NE - 1)

    xs = _dispatch_sc(pos0, pos1, xf)
    sw1e = sw1.reshape(2, INTER, DIM)
    sw3e = sw3.reshape(2, INTER, DIM)
    hs0 = _run_sk1(xf, sw1e, sw3e, 0)
    h = _run_k1(be, xs, w1, w3)
    outs = _run_k2(be, h, w2)
    g0, g1 = _combine_sc(pos0, pos1, outs)
    hs1 = _run_sk1(xf, sw1e, sw3e, 1, dep=outs[:8])
    y = _run_sk2(hs0, hs1, sw2, g0, g1, wt)
    return y.reshape(shape)
